# Initial kernel scaffold; baseline (speedup 1.0000x reference)
import jax
import jax.numpy as jnp
from jax.experimental import pallas as pl


def kernel(indices_tensor, table):
    raise NotImplementedError("write your pallas kernel here")


# speedup vs baseline: 5.1270x; 5.1270x over previous
import functools

import jax
import jax.numpy as jnp
from jax import lax
from jax.experimental import pallas as pl
from jax.experimental.pallas import tpu as pltpu
from jax.experimental.pallas import tpu_sc as plsc

_NC, _NS = 2, 16
_NW = _NC * _NS
_LANE = 128
_K = 8


@functools.partial(jax.jit, static_argnames=())
def _embed_gather(idx2d, table):
    n_rows, _ = idx2d.shape
    depth = table.shape[1]
    rows_per_w = n_rows // _NW
    n_chunks = rows_per_w // _K

    mesh = plsc.VectorSubcoreMesh(core_axis_name="c", subcore_axis_name="s")

    @functools.partial(
        pl.kernel,
        out_type=jax.ShapeDtypeStruct((n_rows, _LANE, depth), jnp.float32),
        mesh=mesh,
        scratch_types=[
            pltpu.VMEM((_K, _LANE), jnp.int32),
            pltpu.VMEM((_K, _LANE, depth), jnp.float32),
            pltpu.SemaphoreType.DMA,
        ],
        compiler_params=pltpu.CompilerParams(use_tc_tiling_on_sc=False),
    )
    def run(idx_hbm, table_hbm, out_hbm, idx_v, rows_v, sem):
        wid = lax.axis_index("s") * _NC + lax.axis_index("c")
        row0 = wid * rows_per_w

        @pl.loop(0, n_chunks)
        def _chunk(g):
            r = row0 + g * _K
            pltpu.sync_copy(idx_hbm.at[pl.ds(r, _K)], idx_v)
            copies = [
                pltpu.async_copy(table_hbm.at[idx_v.at[j]], rows_v.at[j], sem)
                for j in range(_K)
            ]
            for cp in copies:
                cp.wait()
            pltpu.sync_copy(rows_v, out_hbm.at[pl.ds(r, _K)])

    return run(idx2d, table)


def kernel(indices_tensor, table):
    batch, seq = indices_tensor.shape
    depth = table.shape[1]
    idx2d = indices_tensor.astype(jnp.int32).reshape(-1, _LANE)
    out = _embed_gather(idx2d, table)
    return out.reshape(batch, seq, depth)


# speedup vs baseline: 5.2969x; 1.0331x over previous
import functools

import jax
import jax.numpy as jnp
from jax import lax
from jax.experimental import pallas as pl
from jax.experimental.pallas import tpu as pltpu
from jax.experimental.pallas import tpu_sc as plsc

_NC, _NS = 2, 16
_NW = _NC * _NS
_LANE = 128
_K = 8
_NBUF = 2


def _embed_gather(idx2d, table):
    n_rows, _ = idx2d.shape
    depth = table.shape[1]
    rows_per_w = n_rows // _NW
    n_chunks = rows_per_w // _K

    mesh = plsc.VectorSubcoreMesh(core_axis_name="c", subcore_axis_name="s")

    @functools.partial(
        pl.kernel,
        out_type=jax.ShapeDtypeStruct((n_rows, _LANE, depth), jnp.float32),
        mesh=mesh,
        scratch_types=[
            pltpu.VMEM((_NBUF, _K, _LANE), jnp.int32),
            pltpu.VMEM((_NBUF, _K, _LANE, depth), jnp.float32),
            pltpu.SemaphoreType.DMA((_NBUF,)),
            pltpu.SemaphoreType.DMA,
        ],
        compiler_params=pltpu.CompilerParams(use_tc_tiling_on_sc=False),
    )
    def run(idx_hbm, table_hbm, out_hbm, idx_v, rows_v, gsem, osem):
        wid = lax.axis_index("s") * _NC + lax.axis_index("c")
        row0 = wid * rows_per_w

        def stage_and_fire(c, b):
            r = row0 + c * _K
            pltpu.sync_copy(idx_hbm.at[pl.ds(r, _K)], idx_v.at[b])
            for j in range(_K):
                pltpu.async_copy(
                    table_hbm.at[idx_v.at[b].at[j]], rows_v.at[b].at[j],
                    gsem.at[b])

        def wait_gathers(b):
            for j in range(_K):
                pltpu.make_async_copy(
                    table_hbm.at[idx_v.at[b].at[j]], rows_v.at[b].at[j],
                    gsem.at[b]).wait()

        stage_and_fire(0, 0)
        stage_and_fire(1, 1)

        @pl.loop(0, n_chunks)
        def _chunk(c):
            b = lax.rem(c, _NBUF)
            r = row0 + c * _K
            wait_gathers(b)
            out_cp = pltpu.async_copy(rows_v.at[b], out_hbm.at[pl.ds(r, _K)],
                                      osem)
            out_cp.wait()

            @pl.when(c + _NBUF < n_chunks)
            def _next():
                stage_and_fire(c + _NBUF, b)

    return run(idx2d, table)


def kernel(indices_tensor, table):
    batch, seq = indices_tensor.shape
    depth = table.shape[1]
    idx2d = indices_tensor.astype(jnp.int32).reshape(-1, _LANE)
    out = _embed_gather(idx2d, table)
    return out.reshape(batch, seq, depth)
